# baseline (speedup 1.0000x reference)
import functools

import jax
import jax.numpy as jnp
from jax import lax
from jax.experimental import pallas as pl
from jax.experimental.pallas import tpu as pltpu
from jax.experimental.pallas import tpu_sc as plsc

NUM_BAGS = 16
BM = 4096
CH = 128


def _main_kernel(ids_ref, x_ref, w1_ref, b1_ref, w2_ref, b2_ref,
                 top_ref, sums_ref, counts_ref):
    i = pl.program_id(0)
    nb = pl.num_programs(0)
    h = jnp.dot(x_ref[...].astype(jnp.bfloat16), w1_ref[...],
                preferred_element_type=jnp.float32)
    h = jnp.maximum(h + b1_ref[...], 0.0)
    ids = ids_ref[0]
    onehot = (jax.lax.broadcasted_iota(jnp.int32, (NUM_BAGS, BM), 0)
              == ids).astype(jnp.float32)
    part = jnp.dot(onehot, h, preferred_element_type=jnp.float32)
    cnt = jnp.broadcast_to(jnp.sum(onehot, axis=1, keepdims=True),
                           counts_ref.shape)

    @pl.when(i == 0)
    def _init():
        sums_ref[...] = part
        counts_ref[...] = cnt

    @pl.when(i != 0)
    def _acc():
        sums_ref[...] += part
        counts_ref[...] += cnt

    @pl.when(i == nb - 1)
    def _top():
        means = sums_ref[...] / jnp.maximum(counts_ref[:, 0:1], 1.0)
        top = jnp.dot(means, w2_ref[...], preferred_element_type=jnp.float32)
        top_ref[...] = top + b2_ref[...]


def _patch_kernel(full_ref, top_ref, out_ref):
    del full_ref
    out_ref[...] = top_ref[...]


def _make_sc_fill(n, d_out):
    info = plsc.get_sparse_core_info()
    nw = info.num_cores * info.num_subcores
    rpw = n // nw
    nch = rpw // CH
    nvec = d_out // 16
    mesh = plsc.VectorSubcoreMesh(core_axis_name="c", subcore_axis_name="s")

    @functools.partial(
        pl.kernel,
        out_type=jax.ShapeDtypeStruct((n, d_out), jnp.float32),
        mesh=mesh,
        scratch_types=[
            pltpu.VMEM((d_out,), jnp.float32),
            pltpu.VMEM((CH, d_out), jnp.float32),
            pltpu.SemaphoreType.DMA,
        ],
    )
    def sc_fill(b2_hbm, out_hbm, b2_v, chunk, sem):
        wid = lax.axis_index("s") * info.num_cores + lax.axis_index("c")
        base = wid * rpw
        pltpu.sync_copy(b2_hbm, b2_v)
        vs = [b2_v[pl.ds(16 * v, 16)] for v in range(nvec)]

        def fill_row(r, carry):
            for v in range(nvec):
                chunk[r, pl.ds(16 * v, 16)] = vs[v]
            return carry

        lax.fori_loop(0, CH, fill_row, 0)
        copies = [
            pltpu.async_copy(chunk, out_hbm.at[pl.ds(base + j * CH, CH)], sem)
            for j in range(nch)
        ]
        for c in copies:
            c.wait()

    return sc_fill


def kernel(x, ids, W1, b1, W2, b2):
    n, d = x.shape
    d_out = W2.shape[1]
    nb = n // BM
    ids3 = ids.reshape(nb, 1, BM)
    b1r = b1.reshape(1, d)
    b2r = b2.reshape(1, d_out)
    w1b = W1.astype(jnp.bfloat16)

    filled = _make_sc_fill(n, d_out)(b2)

    top = pl.pallas_call(
        _main_kernel,
        grid=(nb,),
        in_specs=[
            pl.BlockSpec((1, 1, BM), lambda i: (i, 0, 0)),
            pl.BlockSpec((BM, d), lambda i: (i, 0)),
            pl.BlockSpec((d, d), lambda i: (0, 0)),
            pl.BlockSpec((1, d), lambda i: (0, 0)),
            pl.BlockSpec((d, d_out), lambda i: (0, 0)),
            pl.BlockSpec((1, d_out), lambda i: (0, 0)),
        ],
        out_specs=pl.BlockSpec((NUM_BAGS, d_out), lambda i: (0, 0)),
        out_shape=jax.ShapeDtypeStruct((NUM_BAGS, d_out), jnp.float32),
        scratch_shapes=[
            pltpu.VMEM((NUM_BAGS, d), jnp.float32),
            pltpu.VMEM((NUM_BAGS, 128), jnp.float32),
        ],
    )(ids3, x, w1b, b1r, W2, b2r)

    out = pl.pallas_call(
        _patch_kernel,
        grid=(1,),
        in_specs=[
            pl.BlockSpec((NUM_BAGS, d_out), lambda i: (0, 0)),
            pl.BlockSpec((NUM_BAGS, d_out), lambda i: (0, 0)),
        ],
        out_specs=pl.BlockSpec((NUM_BAGS, d_out), lambda i: (0, 0)),
        out_shape=jax.ShapeDtypeStruct((n, d_out), jnp.float32),
        input_output_aliases={0: 0},
    )(filled, top)
    return out


# speedup vs baseline: 1.3785x; 1.3785x over previous
import jax
import jax.numpy as jnp
from jax.experimental import pallas as pl
from jax.experimental.pallas import tpu as pltpu

NUM_BAGS = 16
BM = 4096


def _main_kernel(ids_ref, x_ref, w1_ref, b1_ref, b2_ref,
                 out_ref, sums_out, counts_out, sums_ref, counts_ref):
    i = pl.program_id(0)
    nb = pl.num_programs(0)
    h = jnp.dot(x_ref[...].astype(jnp.bfloat16), w1_ref[...],
                preferred_element_type=jnp.float32)
    h = jnp.maximum(h + b1_ref[...], 0.0)
    ids = ids_ref[0]
    onehot = (jax.lax.broadcasted_iota(jnp.int32, (NUM_BAGS, BM), 0)
              == ids).astype(jnp.float32)
    part = jnp.dot(onehot, h, preferred_element_type=jnp.float32)
    cnt = jnp.broadcast_to(jnp.sum(onehot, axis=1, keepdims=True),
                           counts_ref.shape)

    @pl.when(i == 0)
    def _init():
        sums_ref[...] = part
        counts_ref[...] = cnt

    @pl.when(i != 0)
    def _acc():
        sums_ref[...] += part
        counts_ref[...] += cnt

    out_ref[...] = jnp.broadcast_to(b2_ref[...], out_ref.shape)

    @pl.when(i == nb - 1)
    def _emit():
        sums_out[...] = sums_ref[...]
        counts_out[...] = counts_ref[...]


def _patch_kernel(full_ref, sums_ref, counts_ref, w2_ref, b2_ref, out_ref):
    del full_ref
    means = sums_ref[...] / jnp.maximum(counts_ref[:, 0:1], 1.0)
    top = jnp.dot(means, w2_ref[...], preferred_element_type=jnp.float32)
    out_ref[...] = top + b2_ref[...]


def kernel(x, ids, W1, b1, W2, b2):
    n, d = x.shape
    d_out = W2.shape[1]
    nb = n // BM
    ids3 = ids.reshape(nb, 1, BM)
    b1r = b1.reshape(1, d)
    b2r = b2.reshape(1, d_out)
    w1b = W1.astype(jnp.bfloat16)

    filled, sums, counts = pl.pallas_call(
        _main_kernel,
        grid=(nb,),
        in_specs=[
            pl.BlockSpec((1, 1, BM), lambda i: (i, 0, 0)),
            pl.BlockSpec((BM, d), lambda i: (i, 0)),
            pl.BlockSpec((d, d), lambda i: (0, 0)),
            pl.BlockSpec((1, d), lambda i: (0, 0)),
            pl.BlockSpec((1, d_out), lambda i: (0, 0)),
        ],
        out_specs=[
            pl.BlockSpec((BM, d_out), lambda i: (i, 0)),
            pl.BlockSpec((NUM_BAGS, d), lambda i: (0, 0)),
            pl.BlockSpec((NUM_BAGS, 128), lambda i: (0, 0)),
        ],
        out_shape=[
            jax.ShapeDtypeStruct((n, d_out), jnp.float32),
            jax.ShapeDtypeStruct((NUM_BAGS, d), jnp.float32),
            jax.ShapeDtypeStruct((NUM_BAGS, 128), jnp.float32),
        ],
        scratch_shapes=[
            pltpu.VMEM((NUM_BAGS, d), jnp.float32),
            pltpu.VMEM((NUM_BAGS, 128), jnp.float32),
        ],
    )(ids3, x, w1b, b1r, b2r)

    out = pl.pallas_call(
        _patch_kernel,
        grid=(1,),
        in_specs=[
            pl.BlockSpec((NUM_BAGS, d_out), lambda i: (0, 0)),
            pl.BlockSpec((NUM_BAGS, d), lambda i: (0, 0)),
            pl.BlockSpec((NUM_BAGS, 128), lambda i: (0, 0)),
            pl.BlockSpec((d, d_out), lambda i: (0, 0)),
            pl.BlockSpec((1, d_out), lambda i: (0, 0)),
        ],
        out_specs=pl.BlockSpec((NUM_BAGS, d_out), lambda i: (0, 0)),
        out_shape=jax.ShapeDtypeStruct((n, d_out), jnp.float32),
        input_output_aliases={0: 0},
    )(filled, sums, counts, W2, b2r)
    return out
